# baseline (speedup 1.0000x reference)
import jax
import jax.numpy as jnp
from jax import lax
from jax.experimental import pallas as pl
from jax.experimental.pallas import tpu as pltpu
from jax.experimental.pallas import tpu_sc as plsc

NUSER = 1000000
NGAME = 100000
BATCH = 16384
EMBED = 64
NC = 2
NS = 16
NW = NC * NS
BPW = BATCH // NW
RS = 8
GRAN = 128
GCH = 128
NGCH = BPW // GCH

CLAMP_U = (NUSER - GRAN) & ~(GRAN - 1)
LIM_U = CLAMP_U + GRAN


def _sc_body(ui_hbm, gi_hbm, ut_hbm, g2_hbm, tu_hbm, w_hbm, b_hbm,
             out_hbm,
             iu_v, ig_v, qg, ring_u, g_pair, tail_u,
             s_t, out_v, w_v, b_v, *sems):
    sems_u = sems[:RS]
    sems_g = sems[RS:]
    wid = lax.axis_index("s") * NC + lax.axis_index("c")
    base = wid * BPW

    pltpu.sync_copy(ui_hbm.at[pl.ds(base, BPW)], iu_v.at[pl.ds(0, BPW)])
    pltpu.sync_copy(gi_hbm.at[pl.ds(base, BPW)], ig_v.at[pl.ds(0, BPW)])
    pltpu.sync_copy(w_hbm, w_v)
    pltpu.sync_copy(b_hbm, b_v)
    pltpu.sync_copy(tu_hbm, tail_u)

    for j in range(BPW // 16):
        qg[j // 8, pl.ds((j % 8) * 16, 16)] = ig_v[pl.ds(j * 16, 16)] >> 1

    def fire_g(c):
        return pltpu.async_copy(g2_hbm.at[qg.at[c]], g_pair.at[c % 2],
                                sems_g[c % 2])

    def drain_g(c):
        pltpu.make_async_copy(g2_hbm.at[qg.at[c]], g_pair.at[c % 2],
                              sems_g[c % 2]).wait()

    def offs(r):
        su = iu_v[pl.ds(r, 16)][0]
        ou = jnp.minimum((su >> 7) * GRAN, CLAMP_U)
        return pl.multiple_of(ou, GRAN)

    def fire(r, slot):
        o = offs(r)
        for h in range(2):
            pltpu.async_copy(
                ut_hbm.at[pl.ds(h * 32, 32), pl.ds(o, GRAN)],
                ring_u.at[slot, pl.ds(h * 32, 32)], sems_u[slot])

    def drain(r, slot):
        o = offs(r)
        for h in range(2):
            pltpu.make_async_copy(
                ut_hbm.at[pl.ds(h * 32, 32), pl.ds(o, GRAN)],
                ring_u.at[slot, pl.ds(h * 32, 32)], sems_u[slot]).wait()

    lane = lax.iota(jnp.int32, 16)
    lane_scaled = lane * BPW
    w_regs = [w_v[pl.ds(c * 16, 16)] for c in range(EMBED // 16)]
    c_lanes = [lane + c * 16 for c in range(EMBED // 16)]
    zero16 = jnp.full((16,), 0, jnp.int32)

    fire_g(0)
    fire_g(1)
    for s in range(RS - 1):
        fire(s, s)

    def make_row_block(gc):
        def row_block(i, carry):
            for b in range(RS):
                lr = i * RS + b
                r = gc * GCH + lr
                drain(r, b)

                @pl.when(r + (RS - 1) < BPW)
                def _():
                    fire(r + (RS - 1), (b + RS - 1) % RS)

                iu = plsc.load_gather(iu_v, [zero16 + r])
                ig = plsc.load_gather(ig_v, [zero16 + r])
                ru = (iu - jnp.minimum((iu >> 7) * GRAN, CLAMP_U)) & (GRAN - 1)
                tru = (iu - (NUSER - GRAN)) & (GRAN - 1)
                mu = iu >= LIM_U
                gcol = (ig & 1) * EMBED
                s = jnp.zeros((16,), jnp.float32)
                for c in range(EMBED // 16):
                    uc = plsc.load_gather(ring_u.at[b], [c_lanes[c], ru])
                    tuc = plsc.load_gather(tail_u, [c_lanes[c], tru])
                    gc_ = plsc.load_gather(g_pair.at[gc % 2],
                                           [zero16 + lr, gcol + c_lanes[c]])
                    uc = jnp.where(mu, tuc, uc)
                    s = s + uc * gc_ * w_regs[c]
                plsc.store_scatter(s_t, [lane_scaled + r], s)
            return carry
        return row_block

    for gc in range(NGCH):
        drain_g(gc)
        lax.fori_loop(0, GCH // RS, make_row_block(gc), 0)
        if gc + 2 < NGCH:
            fire_g(gc + 2)

    bias = b_v[...]

    def grp_fn(g, carry):
        acc = bias
        for j in range(16):
            acc = acc + s_t[pl.ds(j * BPW + g * 16, 16)]
        out_v[pl.ds(g * 16, 16)] = acc
        return carry

    lax.fori_loop(0, BPW // 16, grp_fn, 0)

    pltpu.sync_copy(out_v, out_hbm.at[pl.ds(base, BPW)])


@jax.jit
def _sc_call(user_index, game_index, emb_user_t, emb_game_pair,
             tail_user_t, w, b16):
    mesh = plsc.VectorSubcoreMesh(core_axis_name="c", subcore_axis_name="s")
    fn = pl.kernel(
        _sc_body,
        out_type=jax.ShapeDtypeStruct((BATCH,), jnp.float32),
        mesh=mesh,
        compiler_params=pltpu.CompilerParams(needs_layout_passes=False),
        scratch_types=(
            [
                pltpu.VMEM((BPW + 16,), jnp.int32),
                pltpu.VMEM((BPW + 16,), jnp.int32),
                pltpu.VMEM((NGCH, GCH), jnp.int32),
                pltpu.VMEM((RS, EMBED, GRAN), jnp.float32),
                pltpu.VMEM((2, GCH, 2 * EMBED), jnp.float32),
                pltpu.VMEM((EMBED, GRAN), jnp.float32),
                pltpu.VMEM((16 * BPW,), jnp.float32),
                pltpu.VMEM((BPW,), jnp.float32),
                pltpu.VMEM((EMBED,), jnp.float32),
                pltpu.VMEM((16,), jnp.float32),
            ]
            + [pltpu.SemaphoreType.DMA] * (RS + 2)
        ),
    )
    return fn(user_index, game_index, emb_user_t, emb_game_pair,
              tail_user_t, w, b16)


def kernel(user_index, game_index, emb_gcf_user, emb_gcf_game, fc_w, fc_b):
    ut = jnp.transpose(emb_gcf_user)
    tu = ut[:, NUSER - GRAN:]
    g2 = emb_gcf_game.reshape(NGAME // 2, 2 * EMBED)
    w = fc_w.reshape(EMBED)
    b16 = jnp.broadcast_to(fc_b, (16,))
    return _sc_call(user_index, game_index, ut, g2, tu, w, b16)


# speedup vs baseline: 1.0045x; 1.0045x over previous
import jax
import jax.numpy as jnp
from jax import lax
from jax.experimental import pallas as pl
from jax.experimental.pallas import tpu as pltpu
from jax.experimental.pallas import tpu_sc as plsc

NUSER = 1000000
NGAME = 100000
BATCH = 16384
EMBED = 64
NC = 2
NS = 16
NW = NC * NS
BPW = BATCH // NW
RS = 8
GRAN = 128
GCH = 128
NGCH = BPW // GCH

CLAMP_U = (NUSER - GRAN) & ~(GRAN - 1)
LIM_U = CLAMP_U + GRAN


def _sc_body(ui_hbm, gi_hbm, ut_hbm, g2_hbm, tu_hbm, w_hbm, b_hbm,
             out_hbm,
             iu_v, ig_v, qg, ring_u, g_pair, tail_u,
             s_t, out_v, w_v, b_v, *sems):
    sems_u = sems[:RS]
    sems_g = sems[RS:]
    wid = lax.axis_index("s") * NC + lax.axis_index("c")
    base = wid * BPW

    pltpu.sync_copy(ui_hbm.at[pl.ds(base, BPW)], iu_v.at[pl.ds(0, BPW)])
    pltpu.sync_copy(gi_hbm.at[pl.ds(base, BPW)], ig_v.at[pl.ds(0, BPW)])
    pltpu.sync_copy(w_hbm, w_v)
    pltpu.sync_copy(b_hbm, b_v)
    pltpu.sync_copy(tu_hbm, tail_u)

    for j in range(BPW // 16):
        qg[j // 8, pl.ds((j % 8) * 16, 16)] = ig_v[pl.ds(j * 16, 16)] >> 1

    def fire_g(c):
        return pltpu.async_copy(g2_hbm.at[qg.at[c]], g_pair.at[c % 2],
                                sems_g[c % 2])

    def drain_g(c):
        pltpu.make_async_copy(g2_hbm.at[qg.at[c]], g_pair.at[c % 2],
                              sems_g[c % 2]).wait()

    def offs(r):
        su = iu_v[pl.ds(r, 16)][0]
        ou = jnp.minimum((su >> 7) * GRAN, CLAMP_U)
        return pl.multiple_of(ou, GRAN)

    def fire(r, slot):
        pltpu.async_copy(ut_hbm.at[:, pl.ds(offs(r), GRAN)], ring_u.at[slot],
                         sems_u[slot])

    def drain(r, slot):
        pltpu.make_async_copy(ut_hbm.at[:, pl.ds(offs(r), GRAN)],
                              ring_u.at[slot], sems_u[slot]).wait()

    lane = lax.iota(jnp.int32, 16)
    lane_scaled = lane * BPW
    w_regs = [w_v[pl.ds(c * 16, 16)] for c in range(EMBED // 16)]
    c_lanes = [lane + c * 16 for c in range(EMBED // 16)]
    zero16 = jnp.full((16,), 0, jnp.int32)

    fire_g(0)
    fire_g(1)
    for s in range(RS - 1):
        fire(s, s)

    def make_row_block(gc):
        def row_block(i, carry):
            for b in range(RS):
                lr = i * RS + b
                r = gc * GCH + lr
                drain(r, b)

                @pl.when(r + (RS - 1) < BPW)
                def _():
                    fire(r + (RS - 1), (b + RS - 1) % RS)

                iu = plsc.load_gather(iu_v, [zero16 + r])
                ig = plsc.load_gather(ig_v, [zero16 + r])
                ru = (iu - jnp.minimum((iu >> 7) * GRAN, CLAMP_U)) & (GRAN - 1)
                tru = (iu - (NUSER - GRAN)) & (GRAN - 1)
                mu = iu >= LIM_U
                gcol = (ig & 1) * EMBED
                s = jnp.zeros((16,), jnp.float32)
                for c in range(EMBED // 16):
                    uc = plsc.load_gather(ring_u.at[b], [c_lanes[c], ru])
                    tuc = plsc.load_gather(tail_u, [c_lanes[c], tru])
                    gc_ = plsc.load_gather(g_pair.at[gc % 2],
                                           [zero16 + lr, gcol + c_lanes[c]])
                    uc = jnp.where(mu, tuc, uc)
                    s = s + uc * gc_ * w_regs[c]
                plsc.store_scatter(s_t, [lane_scaled + r], s)
            return carry
        return row_block

    for gc in range(NGCH):
        drain_g(gc)
        lax.fori_loop(0, GCH // RS, make_row_block(gc), 0)
        if gc + 2 < NGCH:
            fire_g(gc + 2)

    bias = b_v[...]

    def grp_fn(g, carry):
        acc = bias
        for j in range(16):
            acc = acc + s_t[pl.ds(j * BPW + g * 16, 16)]
        out_v[pl.ds(g * 16, 16)] = acc
        return carry

    lax.fori_loop(0, BPW // 16, grp_fn, 0)

    pltpu.sync_copy(out_v, out_hbm.at[pl.ds(base, BPW)])


@jax.jit
def _sc_call(user_index, game_index, emb_user_t, emb_game_pair,
             tail_user_t, w, b16):
    mesh = plsc.VectorSubcoreMesh(core_axis_name="c", subcore_axis_name="s")
    fn = pl.kernel(
        _sc_body,
        out_type=jax.ShapeDtypeStruct((BATCH,), jnp.float32),
        mesh=mesh,
        compiler_params=pltpu.CompilerParams(needs_layout_passes=False),
        scratch_types=(
            [
                pltpu.VMEM((BPW + 16,), jnp.int32),
                pltpu.VMEM((BPW + 16,), jnp.int32),
                pltpu.VMEM((NGCH, GCH), jnp.int32),
                pltpu.VMEM((RS, EMBED, GRAN), jnp.float32),
                pltpu.VMEM((2, GCH, 2 * EMBED), jnp.float32),
                pltpu.VMEM((EMBED, GRAN), jnp.float32),
                pltpu.VMEM((16 * BPW,), jnp.float32),
                pltpu.VMEM((BPW,), jnp.float32),
                pltpu.VMEM((EMBED,), jnp.float32),
                pltpu.VMEM((16,), jnp.float32),
            ]
            + [pltpu.SemaphoreType.DMA] * (RS + 2)
        ),
    )
    return fn(user_index, game_index, emb_user_t, emb_game_pair,
              tail_user_t, w, b16)


def kernel(user_index, game_index, emb_gcf_user, emb_gcf_game, fc_w, fc_b):
    ut = jnp.transpose(emb_gcf_user)
    tu = ut[:, NUSER - GRAN:]
    g2 = emb_gcf_game.reshape(NGAME // 2, 2 * EMBED)
    w = fc_w.reshape(EMBED)
    b16 = jnp.broadcast_to(fc_b, (16,))
    return _sc_call(user_index, game_index, ut, g2, tu, w, b16)
